# Initial kernel scaffold; baseline (speedup 1.0000x reference)
import jax
import jax.numpy as jnp
from jax.experimental import pallas as pl


def kernel(x):
    raise NotImplementedError("write your pallas kernel here")


# speedup vs baseline: 11.3517x; 11.3517x over previous
import jax
import jax.numpy as jnp
from jax.experimental import pallas as pl

KNN = 16
NPTS = 4096
ROWS = 256
DIM = 16


def _knn_kernel(xr_ref, xc_ref, dst_ref, src_ref):
    s = pl.program_id(0)
    r = pl.program_id(1)
    xr = xr_ref[0]
    xc = xc_ref[0]

    x2r = jnp.sum(xr * xr, axis=1, keepdims=True)
    sq_c = xc * xc
    ones = jnp.ones((8, DIM), jnp.float32)
    x2c_row = jax.lax.dot_general(
        ones, sq_c, (((1,), (1,)), ((), ())),
        preferred_element_type=jnp.float32,
        precision=jax.lax.Precision.HIGHEST)
    x2c = x2c_row[0:1, :]

    mm = jax.lax.dot_general(
        xr, xc, (((1,), (1,)), ((), ())),
        preferred_element_type=jnp.float32,
        precision=jax.lax.Precision.DEFAULT)

    dist = (x2r + x2c) - 2.0 * mm
    iota = jax.lax.broadcasted_iota(jnp.int32, (ROWS, NPTS), 1)
    offset = s * NPTS
    for k in range(KNN):
        m = jnp.min(dist, axis=1, keepdims=True)
        idx = jnp.min(jnp.where(dist == m, iota, NPTS), axis=1)
        dst_ref[0, k, :] = idx + offset
        dist = jnp.where(iota == idx[:, None], jnp.float32(jnp.inf), dist)

    row_ids = jax.lax.broadcasted_iota(jnp.int32, (KNN, ROWS), 1)
    src_ref[0] = row_ids + (r * ROWS + offset)


def kernel(x):
    n_samples, n_points, dim = x.shape
    grid = (n_samples, n_points // ROWS)
    out_shape = jax.ShapeDtypeStruct((n_samples, KNN, n_points), jnp.int32)
    dst_t, src_t = pl.pallas_call(
        _knn_kernel,
        grid=grid,
        in_specs=[
            pl.BlockSpec((1, ROWS, dim), lambda s, r: (s, r, 0)),
            pl.BlockSpec((1, n_points, dim), lambda s, r: (s, 0, 0)),
        ],
        out_specs=[
            pl.BlockSpec((1, KNN, ROWS), lambda s, r: (s, 0, r)),
            pl.BlockSpec((1, KNN, ROWS), lambda s, r: (s, 0, r)),
        ],
        out_shape=[out_shape, out_shape],
    )(x, x)
    dst = dst_t.transpose(0, 2, 1).reshape(-1)
    src = src_t.transpose(0, 2, 1).reshape(-1)
    return src, dst
